# Initial kernel scaffold; baseline (speedup 1.0000x reference)
import jax
import jax.numpy as jnp
from jax.experimental import pallas as pl


def kernel(inputs, pos_table):
    raise NotImplementedError("write your pallas kernel here")


# speedup vs baseline: 1.6678x; 1.6678x over previous
import jax
import jax.numpy as jnp
from jax.experimental import pallas as pl

_SEQ_BLK = 512


def _add_kernel(x_ref, t_ref, o_ref):
    o_ref[...] = x_ref[...] + t_ref[...]


def kernel(inputs, pos_table):
    B, S, D = inputs.shape
    ns = S // _SEQ_BLK
    return pl.pallas_call(
        _add_kernel,
        grid=(ns, B),
        in_specs=[
            pl.BlockSpec((1, _SEQ_BLK, D), lambda s, b: (b, s, 0)),
            pl.BlockSpec((_SEQ_BLK, D), lambda s, b: (s, 0)),
        ],
        out_specs=pl.BlockSpec((1, _SEQ_BLK, D), lambda s, b: (b, s, 0)),
        out_shape=jax.ShapeDtypeStruct(inputs.shape, inputs.dtype),
    )(inputs, pos_table)


# speedup vs baseline: 1.7319x; 1.0385x over previous
import jax
import jax.numpy as jnp
from jax.experimental import pallas as pl

_SEQ_BLK = 1024


def _add_kernel(x_ref, t_ref, o_ref):
    o_ref[...] = x_ref[...] + t_ref[...]


def kernel(inputs, pos_table):
    B, S, D = inputs.shape
    ns = S // _SEQ_BLK
    return pl.pallas_call(
        _add_kernel,
        grid=(ns, B),
        in_specs=[
            pl.BlockSpec((1, _SEQ_BLK, D), lambda s, b: (b, s, 0)),
            pl.BlockSpec((_SEQ_BLK, D), lambda s, b: (s, 0)),
        ],
        out_specs=pl.BlockSpec((1, _SEQ_BLK, D), lambda s, b: (b, s, 0)),
        out_shape=jax.ShapeDtypeStruct(inputs.shape, inputs.dtype),
    )(inputs, pos_table)


# speedup vs baseline: 1.7332x; 1.0008x over previous
import jax
import jax.numpy as jnp
from jax.experimental import pallas as pl
from jax.experimental.pallas import tpu as pltpu

_SEQ_BLK = 1024


def _add_kernel(x_ref, t_ref, o_ref):
    o_ref[...] = x_ref[...] + t_ref[...]


def kernel(inputs, pos_table):
    B, S, D = inputs.shape
    ns = S // _SEQ_BLK
    return pl.pallas_call(
        _add_kernel,
        grid=(ns, B),
        in_specs=[
            pl.BlockSpec((1, _SEQ_BLK, D), lambda s, b: (b, s, 0)),
            pl.BlockSpec((_SEQ_BLK, D), lambda s, b: (s, 0)),
        ],
        out_specs=pl.BlockSpec((1, _SEQ_BLK, D), lambda s, b: (b, s, 0)),
        out_shape=jax.ShapeDtypeStruct(inputs.shape, inputs.dtype),
        compiler_params=pltpu.CompilerParams(
            dimension_semantics=("parallel", "arbitrary"),
        ),
    )(inputs, pos_table)
